# baseline (speedup 1.0000x reference)
import jax
import jax.numpy as jnp
import numpy as np
from jax.experimental import pallas as pl
from jax.experimental.pallas import tpu as pltpu

NUM_THETAS = 32
BUMP_STEPS = 32
NUM_FEATURES = 128
R = 1.1
SCALE = 8.0
NG = 64
BLK = 4096
HALF = BLK // 2
PAD_B = 2 * NG
ST = BUMP_STEPS * NUM_THETAS

_LIN = np.linspace(-R, R, BUMP_STEPS, dtype=np.float32)
_E = [float(v) for v in np.exp(np.float64(SCALE) * _LIN)]


def _fused(batch_ref, x_ref, vs_ref, out_ref, cnt_ref):
    i = pl.program_id(0)

    @pl.when(i == 0)
    def _init():
        out_ref[...] = jnp.zeros_like(out_ref)
        cnt_ref[...] = jnp.zeros_like(cnt_ref)

    x = x_ref[...]
    vs = vs_ref[...]
    nh = jax.lax.dot_general(
        vs, x, (((1,), (1,)), ((), ())), preferred_element_type=jnp.float32
    )
    p = jnp.exp(nh).astype(jnp.bfloat16)
    q = jnp.concatenate(
        [1.0 / (p + jnp.bfloat16(_E[s])) for s in range(BUMP_STEPS)],
        axis=0,
    )

    b = batch_ref[0, 0, :]
    half = jax.lax.broadcasted_iota(jnp.int32, (1, BLK), 1) >= HALF
    b2 = b[None, :] + jnp.where(half, NG, 0)
    g = jax.lax.broadcasted_iota(jnp.int32, (2 * NG, BLK), 0)
    onehot = (g == b2).astype(jnp.bfloat16)
    contrib = jax.lax.dot_general(
        q, onehot, (((1,), (1,)), ((), ())), preferred_element_type=jnp.float32,
    )
    out_ref[...] += contrib
    ones = jnp.ones((8, BLK), dtype=jnp.bfloat16)
    cnt = jax.lax.dot_general(
        ones, onehot, (((1,), (1,)), ((), ())),
        preferred_element_type=jnp.float32,
    )
    cnt_ref[...] += cnt


def kernel(x, batch, num_graphs, v):
    del num_graphs
    n = x.shape[0]
    nblocks = (n + BLK - 1) // BLK
    npad = nblocks * BLK - n
    if npad:
        x = jnp.pad(x, ((0, npad), (0, 0)))
        batch = jnp.pad(batch, (0, npad), constant_values=PAD_B)
    batch3 = batch.reshape(nblocks, 1, BLK)
    x = x.astype(jnp.bfloat16)
    vs = (SCALE * v).astype(jnp.bfloat16)

    out, cnt = pl.pallas_call(
        _fused,
        grid=(nblocks,),
        in_specs=[
            pl.BlockSpec((1, 1, BLK), lambda i: (i, 0, 0)),
            pl.BlockSpec((BLK, NUM_FEATURES), lambda i: (i, 0)),
            pl.BlockSpec((NUM_THETAS, NUM_FEATURES), lambda i: (0, 0)),
        ],
        out_specs=[
            pl.BlockSpec((ST, 2 * NG), lambda i: (0, 0)),
            pl.BlockSpec((8, 2 * NG), lambda i: (0, 0)),
        ],
        out_shape=[
            jax.ShapeDtypeStruct((ST, 2 * NG), jnp.float32),
            jax.ShapeDtypeStruct((8, 2 * NG), jnp.float32),
        ],
        compiler_params=pltpu.CompilerParams(
            dimension_semantics=("arbitrary",),
        ),
    )(batch3, x, vs)

    out = out[:, :NG] + out[:, NG:]
    counts = cnt[0, :NG] + cnt[0, NG:]
    lin = jnp.asarray(_LIN)
    e_col = jnp.repeat(jnp.exp(SCALE * lin), NUM_THETAS)[:, None]
    c_row = jnp.repeat(jax.nn.sigmoid(SCALE * (lin - R)), NUM_THETAS)[None, :]
    res = (out * e_col).T - counts[:, None] * c_row
    return res.reshape(NG, BUMP_STEPS, NUM_THETAS)


# speedup vs baseline: 1.2171x; 1.2171x over previous
import jax
import jax.numpy as jnp
import numpy as np
from jax.experimental import pallas as pl
from jax.experimental.pallas import tpu as pltpu

NUM_THETAS = 32
BUMP_STEPS = 32
NUM_FEATURES = 128
R = 1.1
SCALE = 8.0
NG = 64
BLK = 6272
ST = BUMP_STEPS * NUM_THETAS

_LIN = np.linspace(-R, R, BUMP_STEPS, dtype=np.float32)
_E = [float(v) for v in np.exp(np.float64(SCALE) * _LIN)]


def _fused(batch_ref, x_ref, vs_ref, out_ref, cnt_ref):
    i = pl.program_id(0)

    @pl.when(i == 0)
    def _init():
        out_ref[...] = jnp.zeros_like(out_ref)
        cnt_ref[...] = jnp.zeros_like(cnt_ref)

    x = x_ref[...]
    vs = vs_ref[...]
    nh = jax.lax.dot_general(
        vs, x, (((1,), (1,)), ((), ())), preferred_element_type=jnp.float32
    )
    p = jnp.exp(nh).astype(jnp.bfloat16)
    q = jnp.concatenate(
        [1.0 / (p + jnp.bfloat16(_E[s])) for s in range(BUMP_STEPS)],
        axis=0,
    )

    b = batch_ref[0, 0, :]
    g = jax.lax.broadcasted_iota(jnp.int32, (NG, BLK), 0)
    onehot = (g == b[None, :]).astype(jnp.bfloat16)
    contrib = jax.lax.dot_general(
        q, onehot, (((1,), (1,)), ((), ())), preferred_element_type=jnp.float32,
    )
    out_ref[...] += contrib
    ones = jnp.ones((8, BLK), dtype=jnp.bfloat16)
    cnt = jax.lax.dot_general(
        ones, onehot, (((1,), (1,)), ((), ())),
        preferred_element_type=jnp.float32,
    )
    cnt_ref[...] += cnt


def kernel(x, batch, num_graphs, v):
    del num_graphs
    n = x.shape[0]
    nblocks = (n + BLK - 1) // BLK
    npad = nblocks * BLK - n
    if npad:
        x = jnp.pad(x, ((0, npad), (0, 0)))
        batch = jnp.pad(batch, (0, npad), constant_values=NG)
    batch3 = batch.reshape(nblocks, 1, BLK)
    x = x.astype(jnp.bfloat16)
    vs = (SCALE * v).astype(jnp.bfloat16)

    out, cnt = pl.pallas_call(
        _fused,
        grid=(nblocks,),
        in_specs=[
            pl.BlockSpec((1, 1, BLK), lambda i: (i, 0, 0)),
            pl.BlockSpec((BLK, NUM_FEATURES), lambda i: (i, 0)),
            pl.BlockSpec((NUM_THETAS, NUM_FEATURES), lambda i: (0, 0)),
        ],
        out_specs=[
            pl.BlockSpec((ST, NG), lambda i: (0, 0)),
            pl.BlockSpec((8, NG), lambda i: (0, 0)),
        ],
        out_shape=[
            jax.ShapeDtypeStruct((ST, NG), jnp.float32),
            jax.ShapeDtypeStruct((8, NG), jnp.float32),
        ],
        compiler_params=pltpu.CompilerParams(
            dimension_semantics=("arbitrary",),
        ),
    )(batch3, x, vs)

    lin = jnp.asarray(_LIN)
    e_col = jnp.repeat(jnp.exp(SCALE * lin), NUM_THETAS)[:, None]
    c_row = jnp.repeat(jax.nn.sigmoid(SCALE * (lin - R)), NUM_THETAS)[None, :]
    res = (out * e_col).T - cnt[0][None, :].T * c_row
    return res.reshape(NG, BUMP_STEPS, NUM_THETAS)


# speedup vs baseline: 1.2283x; 1.0092x over previous
import jax
import jax.numpy as jnp
import numpy as np
from jax.experimental import pallas as pl
from jax.experimental.pallas import tpu as pltpu

NUM_THETAS = 32
BUMP_STEPS = 32
NUM_FEATURES = 128
R = 1.1
SCALE = 8.0
NG = 64
BLK = 12544
ST = BUMP_STEPS * NUM_THETAS

_LIN = np.linspace(-R, R, BUMP_STEPS, dtype=np.float32)
_E = [float(v) for v in np.exp(np.float64(SCALE) * _LIN)]


def _fused(batch_ref, x_ref, vs_ref, out_ref, cnt_ref):
    i = pl.program_id(0)

    @pl.when(i == 0)
    def _init():
        out_ref[...] = jnp.zeros_like(out_ref)
        cnt_ref[...] = jnp.zeros_like(cnt_ref)

    x = x_ref[...]
    vs = vs_ref[...]
    nh = jax.lax.dot_general(
        vs, x, (((1,), (1,)), ((), ())), preferred_element_type=jnp.float32
    )
    p = jnp.exp(nh).astype(jnp.bfloat16)
    q = jnp.concatenate(
        [1.0 / (p + jnp.bfloat16(_E[s])) for s in range(BUMP_STEPS)],
        axis=0,
    )

    b = batch_ref[0, 0, :]
    g = jax.lax.broadcasted_iota(jnp.int32, (NG, BLK), 0)
    onehot = (g == b[None, :]).astype(jnp.bfloat16)
    contrib = jax.lax.dot_general(
        q, onehot, (((1,), (1,)), ((), ())), preferred_element_type=jnp.float32,
    )
    out_ref[...] += contrib
    ones = jnp.ones((8, BLK), dtype=jnp.bfloat16)
    cnt = jax.lax.dot_general(
        ones, onehot, (((1,), (1,)), ((), ())),
        preferred_element_type=jnp.float32,
    )
    cnt_ref[...] += cnt


def kernel(x, batch, num_graphs, v):
    del num_graphs
    n = x.shape[0]
    nblocks = (n + BLK - 1) // BLK
    npad = nblocks * BLK - n
    if npad:
        x = jnp.pad(x, ((0, npad), (0, 0)))
        batch = jnp.pad(batch, (0, npad), constant_values=NG)
    batch3 = batch.reshape(nblocks, 1, BLK)
    x = x.astype(jnp.bfloat16)
    vs = (SCALE * v).astype(jnp.bfloat16)

    out, cnt = pl.pallas_call(
        _fused,
        grid=(nblocks,),
        in_specs=[
            pl.BlockSpec((1, 1, BLK), lambda i: (i, 0, 0)),
            pl.BlockSpec((BLK, NUM_FEATURES), lambda i: (i, 0)),
            pl.BlockSpec((NUM_THETAS, NUM_FEATURES), lambda i: (0, 0)),
        ],
        out_specs=[
            pl.BlockSpec((ST, NG), lambda i: (0, 0)),
            pl.BlockSpec((8, NG), lambda i: (0, 0)),
        ],
        out_shape=[
            jax.ShapeDtypeStruct((ST, NG), jnp.float32),
            jax.ShapeDtypeStruct((8, NG), jnp.float32),
        ],
        compiler_params=pltpu.CompilerParams(
            dimension_semantics=("arbitrary",),
        ),
    )(batch3, x, vs)

    lin = jnp.asarray(_LIN)
    e_col = jnp.repeat(jnp.exp(SCALE * lin), NUM_THETAS)[:, None]
    c_row = jnp.repeat(jax.nn.sigmoid(SCALE * (lin - R)), NUM_THETAS)[None, :]
    res = (out * e_col).T - cnt[0][None, :].T * c_row
    return res.reshape(NG, BUMP_STEPS, NUM_THETAS)


# speedup vs baseline: 1.2285x; 1.0002x over previous
import jax
import jax.numpy as jnp
import numpy as np
from jax.experimental import pallas as pl
from jax.experimental.pallas import tpu as pltpu

NUM_THETAS = 32
BUMP_STEPS = 32
NUM_FEATURES = 128
R = 1.1
SCALE = 8.0
NG = 64
BLK = 12544
ST = BUMP_STEPS * NUM_THETAS

_LIN = np.linspace(-R, R, BUMP_STEPS, dtype=np.float32)
_E = [float(v) for v in np.exp(np.float64(SCALE) * _LIN)]


def _fused(batch_ref, x_ref, vs_ref, out_ref, cnt_ref):
    i = pl.program_id(0)

    @pl.when(i == 0)
    def _init():
        out_ref[...] = jnp.zeros_like(out_ref)
        cnt_ref[...] = jnp.zeros_like(cnt_ref)

    x = x_ref[...]
    vs = vs_ref[...]
    nh = jax.lax.dot_general(
        vs, x, (((1,), (1,)), ((), ())), preferred_element_type=jnp.float32
    )
    p = jnp.exp(nh).astype(jnp.bfloat16)

    b = batch_ref[0, 0, :]
    g = jax.lax.broadcasted_iota(jnp.int32, (NG, BLK), 0)
    onehot = (g == b[None, :]).astype(jnp.bfloat16)

    for grp in range(BUMP_STEPS // 4):
        q = jnp.concatenate(
            [
                1.0 / (p + jnp.bfloat16(_E[s]))
                for s in range(4 * grp, 4 * grp + 4)
            ],
            axis=0,
        )
        contrib = jax.lax.dot_general(
            q, onehot, (((1,), (1,)), ((), ())),
            preferred_element_type=jnp.float32,
        )
        out_ref[pl.ds(128 * grp, 128), :] += contrib

    ones = jnp.ones((8, BLK), dtype=jnp.bfloat16)
    cnt = jax.lax.dot_general(
        ones, onehot, (((1,), (1,)), ((), ())),
        preferred_element_type=jnp.float32,
    )
    cnt_ref[...] += cnt


def kernel(x, batch, num_graphs, v):
    del num_graphs
    n = x.shape[0]
    nblocks = (n + BLK - 1) // BLK
    npad = nblocks * BLK - n
    if npad:
        x = jnp.pad(x, ((0, npad), (0, 0)))
        batch = jnp.pad(batch, (0, npad), constant_values=NG)
    batch3 = batch.reshape(nblocks, 1, BLK)
    x = x.astype(jnp.bfloat16)
    vs = (SCALE * v).astype(jnp.bfloat16)

    out, cnt = pl.pallas_call(
        _fused,
        grid=(nblocks,),
        in_specs=[
            pl.BlockSpec((1, 1, BLK), lambda i: (i, 0, 0)),
            pl.BlockSpec((BLK, NUM_FEATURES), lambda i: (i, 0)),
            pl.BlockSpec((NUM_THETAS, NUM_FEATURES), lambda i: (0, 0)),
        ],
        out_specs=[
            pl.BlockSpec((ST, NG), lambda i: (0, 0)),
            pl.BlockSpec((8, NG), lambda i: (0, 0)),
        ],
        out_shape=[
            jax.ShapeDtypeStruct((ST, NG), jnp.float32),
            jax.ShapeDtypeStruct((8, NG), jnp.float32),
        ],
        compiler_params=pltpu.CompilerParams(
            dimension_semantics=("arbitrary",),
        ),
    )(batch3, x, vs)

    lin = jnp.asarray(_LIN)
    e_col = jnp.repeat(jnp.exp(SCALE * lin), NUM_THETAS)[:, None]
    c_row = jnp.repeat(jax.nn.sigmoid(SCALE * (lin - R)), NUM_THETAS)[None, :]
    res = (out * e_col).T - cnt[0][None, :].T * c_row
    return res.reshape(NG, BUMP_STEPS, NUM_THETAS)
